# speedup vs baseline: 1.6419x; 1.6419x over previous
import jax
import jax.numpy as jnp
from jax.experimental import pallas as pl
from jax.experimental.pallas import tpu as pltpu

NUM_EXPERTS = 16
TOP_K = 2
NF = 4


def _moe_body(x_ref, gate_ref, w1_ref, w3_ref, w2_ref,
              out_ref, logits_ref,
              w0_ref, w1n_ref, a0_ref, a1_ref):
    e = pl.program_id(0)
    f = pl.program_id(1)

    @pl.when((e == 0) & (f == 0))
    def _router():
        x = x_ref[...]
        logits = jax.lax.dot_general(
            x, gate_ref[...], (((1,), (1,)), ((), ())),
            preferred_element_type=jnp.float32)
        logits_ref[...] = logits
        m = jnp.max(logits, axis=1, keepdims=True)
        p = jnp.exp(logits - m)
        p = p / jnp.sum(p, axis=1, keepdims=True)
        a0 = jnp.argmax(p, axis=1)[:, None]
        cols = jax.lax.broadcasted_iota(jnp.int32, p.shape, 1)
        w0 = jnp.max(p, axis=1, keepdims=True)
        p2 = jnp.where(cols == a0, -jnp.inf, p)
        a1 = jnp.argmax(p2, axis=1)[:, None]
        w1v = jnp.max(p2, axis=1, keepdims=True)
        denom = w0 + w1v
        w0_ref[...] = w0 / denom
        w1n_ref[...] = w1v / denom
        a0_ref[...] = a0.astype(jnp.int32)
        a1_ref[...] = a1.astype(jnp.int32)
        out_ref[...] = jnp.zeros_like(out_ref)

    x = x_ref[...]
    w1b = w1_ref[0]
    w3b = w3_ref[0]
    w2b = w2_ref[0]
    g = jax.lax.dot_general(x, w1b, (((1,), (1,)), ((), ())),
                            preferred_element_type=jnp.float32)
    u = jax.lax.dot_general(x, w3b, (((1,), (1,)), ((), ())),
                            preferred_element_type=jnp.float32)
    h = (g * jax.lax.logistic(g)) * u
    y = jax.lax.dot_general(h, w2b, (((1,), (1,)), ((), ())),
                            preferred_element_type=jnp.float32)
    we = (jnp.where(a0_ref[...] == e, w0_ref[...], 0.0)
          + jnp.where(a1_ref[...] == e, w1n_ref[...], 0.0))
    out_ref[...] += y * we


def kernel(hidden_states, gate_w, w1, w3, w2):
    B, S, H = hidden_states.shape
    E, F, _ = w1.shape
    T = B * S
    FB = F // NF
    x = hidden_states.reshape(T, H)

    out, logits = pl.pallas_call(
        _moe_body,
        grid=(E, NF),
        in_specs=[
            pl.BlockSpec((T, H), lambda e, f: (0, 0)),
            pl.BlockSpec((E, H), lambda e, f: (0, 0)),
            pl.BlockSpec((1, FB, H), lambda e, f: (e, f, 0)),
            pl.BlockSpec((1, FB, H), lambda e, f: (e, f, 0)),
            pl.BlockSpec((1, H, FB), lambda e, f: (e, 0, f)),
        ],
        out_specs=[
            pl.BlockSpec((T, H), lambda e, f: (0, 0)),
            pl.BlockSpec((T, E), lambda e, f: (0, 0)),
        ],
        out_shape=[
            jax.ShapeDtypeStruct((T, H), jnp.float32),
            jax.ShapeDtypeStruct((T, E), jnp.float32),
        ],
        scratch_shapes=[
            pltpu.VMEM((T, 1), jnp.float32),
            pltpu.VMEM((T, 1), jnp.float32),
            pltpu.VMEM((T, 1), jnp.int32),
            pltpu.VMEM((T, 1), jnp.int32),
        ],
    )(x, gate_w, w1, w3, w2)

    return out.reshape(B, S, H), logits
